# Initial kernel scaffold; baseline (speedup 1.0000x reference)
import jax
import jax.numpy as jnp
from jax.experimental import pallas as pl


def kernel(x, edge_index, W1, b1, W2, b2):
    raise NotImplementedError("write your pallas kernel here")


# speedup vs baseline: 8.9850x; 8.9850x over previous
import functools

import jax
import jax.numpy as jnp
from jax import lax
from jax.experimental import pallas as pl
from jax.experimental.pallas import tpu as pltpu
from jax.experimental.pallas import tpu_sc as plsc

N = 10000
NP = 10240
D = 128
NE = 320000
NC, NS = 2, 16
NW = NC * NS
C = 128
EP = NE + N
K = -(-EP // (NW * C))
E_PAD = NW * K * C
PAD = E_PAD - EP
RPS = NP // NS
R = 1000

_mesh = plsc.VectorSubcoreMesh(core_axis_name="c", subcore_axis_name="s")


DG = 9
DGROUPS = K // DG


@functools.partial(
    pl.kernel, mesh=_mesh,
    out_type=jax.ShapeDtypeStruct((NC, NP, D), jnp.float32),
    scratch_types=[
        pltpu.VMEM((DG, C), jnp.int32),
        pltpu.VMEM((DG, C), jnp.int32),
        pltpu.VMEM((C, D), jnp.float32),
        pltpu.VMEM((C, D), jnp.float32),
        pltpu.VMEM_SHARED((NP, D), jnp.float32),
    ],
)
def _sc_degrees(src_hbm, dst_hbm, vals_hbm, z_hbm, deg_hbm,
                src_v, dst_v, val_a, val_b, acc):
    cid = lax.axis_index("c")
    sid = lax.axis_index("s")
    wid = cid * NS + sid
    pltpu.sync_copy(vals_hbm.at[0], val_a)
    pltpu.sync_copy(vals_hbm.at[1], val_b)
    r0 = sid * RPS
    pltpu.sync_copy(z_hbm.at[pl.ds(r0, RPS)], acc.at[pl.ds(r0, RPS)])
    plsc.subcore_barrier()

    @pl.loop(0, DGROUPS)
    def _(g):
        pltpu.sync_copy(src_hbm.at[wid, g], src_v)
        pltpu.sync_copy(dst_hbm.at[wid, g], dst_v)

        @pl.loop(0, DG)
        def _(j):
            pltpu.sync_copy(val_a, acc.at[src_v.at[j]], add=True)
            pltpu.sync_copy(val_b, acc.at[dst_v.at[j]], add=True)

    plsc.subcore_barrier()
    pltpu.sync_copy(acc.at[pl.ds(r0, RPS)], deg_hbm.at[cid, pl.ds(r0, RPS)])


@functools.partial(
    pl.kernel, mesh=_mesh,
    out_type=jax.ShapeDtypeStruct((NC, NP, D), jnp.float32),
    scratch_types=[
        pltpu.VMEM((K, C), jnp.int32),
        pltpu.VMEM((K, C), jnp.int32),
        pltpu.VMEM((C, D), jnp.float32),
        pltpu.VMEM_SHARED((NP, D), jnp.float32),
        pltpu.SemaphoreType.DMA,
    ],
)
def _sc_aggregate(h_hbm, src_hbm, dst_hbm, z_hbm, agg_hbm,
                  src_v, dst_v, rows_v, acc, sem):
    cid = lax.axis_index("c")
    sid = lax.axis_index("s")
    wid = cid * NS + sid
    pltpu.sync_copy(src_hbm.at[wid], src_v)
    pltpu.sync_copy(dst_hbm.at[wid], dst_v)
    r0 = sid * RPS
    pltpu.sync_copy(z_hbm.at[pl.ds(r0, RPS)], acc.at[pl.ds(r0, RPS)])
    plsc.subcore_barrier()

    @pl.loop(0, K)
    def _(j):
        pltpu.async_copy(h_hbm.at[src_v.at[j]], rows_v, sem).wait()
        pltpu.sync_copy(rows_v, acc.at[dst_v.at[j]], add=True)

    plsc.subcore_barrier()
    pltpu.sync_copy(acc.at[pl.ds(r0, RPS)], agg_hbm.at[cid, pl.ds(r0, RPS)])


def _norms(d0, d1):
    od = d0[0, :, 0:1] + d1[0, :, 0:1]
    idg = d0[0, :, 127:128] + d1[0, :, 127:128]
    return (lax.rsqrt(jnp.maximum(od, 1.0)),
            lax.rsqrt(jnp.maximum(idg, 1.0)))


def _tc_pre_body(x_ref, w_ref, d0_ref, d1_ref, h_ref):
    ns, _ = _norms(d0_ref[...], d1_ref[...])
    h_ref[...] = jnp.dot(x_ref[...], w_ref[...],
                         preferred_element_type=jnp.float32) * ns


def _tc_mid_body(a0_ref, a1_ref, d0_ref, d1_ref, b_ref, w_ref, h_ref):
    ns, nd = _norms(d0_ref[...], d1_ref[...])
    agg = a0_ref[0] + a1_ref[0]
    t = jnp.maximum(agg * nd + b_ref[...], 0.0)
    h_ref[...] = jnp.dot(t * ns, w_ref[...],
                         preferred_element_type=jnp.float32)


def _tc_post_body(a0_ref, a1_ref, d0_ref, d1_ref, b_ref, o_ref):
    _, nd = _norms(d0_ref[...], d1_ref[...])
    o_ref[...] = (a0_ref[0] + a1_ref[0]) * nd + b_ref[...]


def _agg_specs():
    return [pl.BlockSpec((1, R, D), lambda i, c=c: (c, i, 0))
            for c in (0, 1)]


_b_spec = pl.BlockSpec((1, D), lambda i: (0, 0))
_w_spec = pl.BlockSpec((D, D), lambda i: (0, 0))
_row_spec = pl.BlockSpec((R, D), lambda i: (i, 0))


def _tc_pre(x, W1, deg):
    return pl.pallas_call(
        _tc_pre_body,
        grid=(N // R,),
        in_specs=[_row_spec, _w_spec] + _agg_specs(),
        out_specs=_row_spec,
        out_shape=jax.ShapeDtypeStruct((N, D), jnp.float32),
    )(x, W1, deg, deg)


def _tc_mid(agg, deg, b1, W2):
    return pl.pallas_call(
        _tc_mid_body,
        grid=(N // R,),
        in_specs=_agg_specs() + _agg_specs() + [_b_spec, _w_spec],
        out_specs=_row_spec,
        out_shape=jax.ShapeDtypeStruct((N, D), jnp.float32),
    )(agg, agg, deg, deg, b1, W2)


def _tc_post(agg, deg, b2):
    return pl.pallas_call(
        _tc_post_body,
        grid=(N // R,),
        in_specs=_agg_specs() + _agg_specs() + [_b_spec],
        out_specs=_row_spec,
        out_shape=jax.ShapeDtypeStruct((N, D), jnp.float32),
    )(agg, agg, deg, deg, b2)


def kernel(x, edge_index, W1, b1, W2, b2):
    loop_idx = jnp.arange(N, dtype=jnp.int32)
    pad_iota = jnp.arange(PAD, dtype=jnp.int32)
    pad_src = pad_iota % N
    pad_dst = N + pad_iota % (NP - N)
    src_all = jnp.concatenate([edge_index[0], loop_idx, pad_src])
    dst_all = jnp.concatenate([edge_index[1], loop_idx, pad_dst])
    src3 = src_all.reshape(NW, K, C)
    dst3 = dst_all.reshape(NW, K, C)
    lane = jnp.arange(D, dtype=jnp.int32)
    vals = jnp.stack([
        jnp.broadcast_to((lane < 64).astype(jnp.float32), (C, D)),
        jnp.broadcast_to((lane >= 64).astype(jnp.float32), (C, D)),
    ])
    z128 = jnp.zeros((NP, D), jnp.float32)

    src4 = src3.reshape(NW, DGROUPS, DG, C)
    dst4 = dst3.reshape(NW, DGROUPS, DG, C)
    deg = _sc_degrees(src4, dst4, vals, z128)
    h1 = _tc_pre(x, W1, deg)
    agg1 = _sc_aggregate(h1, src3, dst3, z128)
    h2 = _tc_mid(agg1, deg, b1.reshape(1, D), W2)
    agg2 = _sc_aggregate(h2, src3, dst3, z128)
    return _tc_post(agg2, deg, b2.reshape(1, D))


# speedup vs baseline: 10.9799x; 1.2220x over previous
import functools

import jax
import jax.numpy as jnp
from jax import lax
from jax.experimental import pallas as pl
from jax.experimental.pallas import tpu as pltpu
from jax.experimental.pallas import tpu_sc as plsc

N = 10000
NP = 10240
D = 128
NE = 320000
NC, NS = 2, 16
NW = NC * NS
C = 128
EP = NE + N
K = 84
G = 14
NG = K // G
E_PAD = NW * K * C
PAD = E_PAD - EP
RPS = NP // NS
R = 1000

_mesh = plsc.VectorSubcoreMesh(core_axis_name="c", subcore_axis_name="s")


DG = G
DGROUPS = K // DG


@functools.partial(
    pl.kernel, mesh=_mesh,
    out_type=jax.ShapeDtypeStruct((NC, NP, D), jnp.float32),
    scratch_types=[
        pltpu.VMEM((DG, C), jnp.int32),
        pltpu.VMEM((DG, C), jnp.int32),
        pltpu.VMEM((C, D), jnp.float32),
        pltpu.VMEM((C, D), jnp.float32),
        pltpu.VMEM_SHARED((NP, D), jnp.float32),
    ],
)
def _sc_degrees(src_hbm, dst_hbm, vals_hbm, z_hbm, deg_hbm,
                src_v, dst_v, val_a, val_b, acc):
    cid = lax.axis_index("c")
    sid = lax.axis_index("s")
    wid = cid * NS + sid
    pltpu.sync_copy(vals_hbm.at[0], val_a)
    pltpu.sync_copy(vals_hbm.at[1], val_b)
    r0 = sid * RPS
    pltpu.sync_copy(z_hbm.at[pl.ds(r0, RPS)], acc.at[pl.ds(r0, RPS)])
    plsc.subcore_barrier()

    @pl.loop(0, DGROUPS)
    def _(g):
        pltpu.sync_copy(src_hbm.at[wid, g], src_v)
        pltpu.sync_copy(dst_hbm.at[wid, g], dst_v)

        @pl.loop(0, DG)
        def _(j):
            pltpu.sync_copy(val_a, acc.at[src_v.at[j]], add=True)
            pltpu.sync_copy(val_b, acc.at[dst_v.at[j]], add=True)

    plsc.subcore_barrier()
    pltpu.sync_copy(acc.at[pl.ds(r0, RPS)], deg_hbm.at[cid, pl.ds(r0, RPS)])


@functools.partial(
    pl.kernel, mesh=_mesh,
    out_type=jax.ShapeDtypeStruct((NC, NP, D), jnp.float32),
    scratch_types=[
        pltpu.VMEM((G, C), jnp.int32),
        pltpu.VMEM((G, C), jnp.int32),
        pltpu.VMEM((C, D), jnp.float32),
        pltpu.VMEM((C, D), jnp.float32),
        pltpu.VMEM_SHARED((NP, D), jnp.float32),
        pltpu.SemaphoreType.DMA,
        pltpu.SemaphoreType.DMA,
    ],
)
def _sc_aggregate(h_hbm, src_hbm, dst_hbm, z_hbm, agg_hbm,
                  src_v, dst_v, rows0, rows1, acc, sem0, sem1):
    cid = lax.axis_index("c")
    sid = lax.axis_index("s")
    wid = cid * NS + sid
    r0 = sid * RPS
    pltpu.sync_copy(z_hbm.at[pl.ds(r0, RPS)], acc.at[pl.ds(r0, RPS)])
    plsc.subcore_barrier()

    @pl.loop(0, NG)
    def _(g):
        pltpu.sync_copy(src_hbm.at[wid, g], src_v)
        pltpu.sync_copy(dst_hbm.at[wid, g], dst_v)
        pltpu.async_copy(h_hbm.at[src_v.at[0]], rows0, sem0)
        for jj in range(G):
            rbuf, rsem = (rows0, sem0) if jj % 2 == 0 else (rows1, sem1)
            nbuf, nsm = (rows1, sem1) if jj % 2 == 0 else (rows0, sem0)
            if jj + 1 < G:
                pltpu.async_copy(h_hbm.at[src_v.at[jj + 1]], nbuf, nsm)
            pltpu.make_async_copy(h_hbm.at[src_v.at[jj]], rbuf, rsem).wait()
            pltpu.sync_copy(rbuf, acc.at[dst_v.at[jj]], add=True)

    plsc.subcore_barrier()
    pltpu.sync_copy(acc.at[pl.ds(r0, RPS)], agg_hbm.at[cid, pl.ds(r0, RPS)])


def _norms(d0, d1):
    od = d0[0, :, 0:1] + d1[0, :, 0:1]
    idg = d0[0, :, 127:128] + d1[0, :, 127:128]
    return (lax.rsqrt(jnp.maximum(od, 1.0)),
            lax.rsqrt(jnp.maximum(idg, 1.0)))


def _tc_pre_body(x_ref, w_ref, d0_ref, d1_ref, h_ref):
    ns, _ = _norms(d0_ref[...], d1_ref[...])
    h_ref[...] = jnp.dot(x_ref[...], w_ref[...],
                         preferred_element_type=jnp.float32) * ns


def _tc_mid_body(a0_ref, a1_ref, d0_ref, d1_ref, b_ref, w_ref, h_ref):
    ns, nd = _norms(d0_ref[...], d1_ref[...])
    agg = a0_ref[0] + a1_ref[0]
    t = jnp.maximum(agg * nd + b_ref[...], 0.0)
    h_ref[...] = jnp.dot(t * ns, w_ref[...],
                         preferred_element_type=jnp.float32)


def _tc_post_body(a0_ref, a1_ref, d0_ref, d1_ref, b_ref, o_ref):
    _, nd = _norms(d0_ref[...], d1_ref[...])
    o_ref[...] = (a0_ref[0] + a1_ref[0]) * nd + b_ref[...]


def _agg_specs():
    return [pl.BlockSpec((1, R, D), lambda i, c=c: (c, i, 0))
            for c in (0, 1)]


_b_spec = pl.BlockSpec((1, D), lambda i: (0, 0))
_w_spec = pl.BlockSpec((D, D), lambda i: (0, 0))
_row_spec = pl.BlockSpec((R, D), lambda i: (i, 0))


def _tc_pre(x, W1, deg):
    return pl.pallas_call(
        _tc_pre_body,
        grid=(N // R,),
        in_specs=[_row_spec, _w_spec] + _agg_specs(),
        out_specs=_row_spec,
        out_shape=jax.ShapeDtypeStruct((N, D), jnp.float32),
    )(x, W1, deg, deg)


def _tc_mid(agg, deg, b1, W2):
    return pl.pallas_call(
        _tc_mid_body,
        grid=(N // R,),
        in_specs=_agg_specs() + _agg_specs() + [_b_spec, _w_spec],
        out_specs=_row_spec,
        out_shape=jax.ShapeDtypeStruct((N, D), jnp.float32),
    )(agg, agg, deg, deg, b1, W2)


def _tc_post(agg, deg, b2):
    return pl.pallas_call(
        _tc_post_body,
        grid=(N // R,),
        in_specs=_agg_specs() + _agg_specs() + [_b_spec],
        out_specs=_row_spec,
        out_shape=jax.ShapeDtypeStruct((N, D), jnp.float32),
    )(agg, agg, deg, deg, b2)


def kernel(x, edge_index, W1, b1, W2, b2):
    loop_idx = jnp.arange(N, dtype=jnp.int32)
    pad_iota = jnp.arange(PAD, dtype=jnp.int32)
    pad_junk = N + pad_iota % (NP - N)
    pad_src_agg = pad_iota % N
    src4d = jnp.concatenate([edge_index[0], loop_idx, pad_junk]
                            ).reshape(NW, NG, G, C)
    src4 = jnp.concatenate([edge_index[0], loop_idx, pad_src_agg]
                           ).reshape(NW, NG, G, C)
    dst4 = jnp.concatenate([edge_index[1], loop_idx, pad_junk]
                           ).reshape(NW, NG, G, C)
    lane = jnp.arange(D, dtype=jnp.int32)
    vals = jnp.stack([
        jnp.broadcast_to((lane < 64).astype(jnp.float32), (C, D)),
        jnp.broadcast_to((lane >= 64).astype(jnp.float32), (C, D)),
    ])
    z128 = jnp.zeros((NP, D), jnp.float32)

    deg = _sc_degrees(src4d, dst4, vals, z128)
    h1 = _tc_pre(x, W1, deg)
    agg1 = _sc_aggregate(h1, src4, dst4, z128)
    h2 = _tc_mid(agg1, deg, b1.reshape(1, D), W2)
    agg2 = _sc_aggregate(h2, src4, dst4, z128)
    return _tc_post(agg2, deg, b2.reshape(1, D))


# speedup vs baseline: 10.9884x; 1.0008x over previous
import functools

import jax
import jax.numpy as jnp
from jax import lax
from jax.experimental import pallas as pl
from jax.experimental.pallas import tpu as pltpu
from jax.experimental.pallas import tpu_sc as plsc

N = 10000
NP = 10240
D = 128
NE = 320000
NC, NS = 2, 16
NW = NC * NS
C = 128
EP = NE + N
K = 84
G = 14
NG = K // G
E_PAD = NW * K * C
PAD = E_PAD - EP
RPS = NP // NS
R = 1000

_mesh = plsc.VectorSubcoreMesh(core_axis_name="c", subcore_axis_name="s")


DG = G
DGROUPS = K // DG


@functools.partial(
    pl.kernel, mesh=_mesh,
    out_type=jax.ShapeDtypeStruct((NC, NP, D), jnp.float32),
    scratch_types=[
        pltpu.VMEM((DG, C), jnp.int32),
        pltpu.VMEM((DG, C), jnp.int32),
        pltpu.VMEM((C, D), jnp.float32),
        pltpu.VMEM((C, D), jnp.float32),
        pltpu.VMEM_SHARED((NP, D), jnp.float32),
    ],
)
def _sc_degrees(src_hbm, dst_hbm, vals_hbm, z_hbm, deg_hbm,
                src_v, dst_v, val_a, val_b, acc):
    cid = lax.axis_index("c")
    sid = lax.axis_index("s")
    wid = cid * NS + sid
    pltpu.sync_copy(vals_hbm.at[0], val_a)
    pltpu.sync_copy(vals_hbm.at[1], val_b)
    r0 = sid * RPS
    pltpu.sync_copy(z_hbm.at[pl.ds(r0, RPS)], acc.at[pl.ds(r0, RPS)])
    plsc.subcore_barrier()

    @pl.loop(0, DGROUPS)
    def _(g):
        pltpu.sync_copy(src_hbm.at[wid, g], src_v)
        pltpu.sync_copy(dst_hbm.at[wid, g], dst_v)

        @pl.loop(0, DG)
        def _(j):
            pltpu.sync_copy(val_a, acc.at[src_v.at[j]], add=True)
            pltpu.sync_copy(val_b, acc.at[dst_v.at[j]], add=True)

    plsc.subcore_barrier()
    pltpu.sync_copy(acc.at[pl.ds(r0, RPS)], deg_hbm.at[cid, pl.ds(r0, RPS)])


@functools.partial(
    pl.kernel, mesh=_mesh,
    out_type=jax.ShapeDtypeStruct((NC, NP, D), jnp.float32),
    scratch_types=[
        pltpu.VMEM((G, C), jnp.int32),
        pltpu.VMEM((G, C), jnp.int32),
        pltpu.VMEM((C, D), jnp.float32),
        pltpu.VMEM((C, D), jnp.float32),
        pltpu.VMEM_SHARED((NP, D), jnp.float32),
        pltpu.SemaphoreType.DMA,
        pltpu.SemaphoreType.DMA,
        pltpu.SemaphoreType.DMA,
        pltpu.SemaphoreType.DMA,
    ],
)
def _sc_aggregate(h_hbm, src_hbm, dst_hbm, z_hbm, agg_hbm,
                  src_v, dst_v, rows0, rows1, acc, sem0, sem1, semS0, semS1):
    cid = lax.axis_index("c")
    sid = lax.axis_index("s")
    wid = cid * NS + sid
    r0 = sid * RPS
    pltpu.sync_copy(z_hbm.at[pl.ds(r0, RPS)], acc.at[pl.ds(r0, RPS)])
    plsc.subcore_barrier()

    @pl.loop(0, NG)
    def _(g):
        pltpu.sync_copy(src_hbm.at[wid, g], src_v)
        pltpu.sync_copy(dst_hbm.at[wid, g], dst_v)
        pltpu.async_copy(h_hbm.at[src_v.at[0]], rows0, sem0)
        for jj in range(G):
            rbuf, gsem, ssem = ((rows0, sem0, semS0) if jj % 2 == 0
                                else (rows1, sem1, semS1))
            nbuf, ngsem, nssem = ((rows1, sem1, semS1) if jj % 2 == 0
                                  else (rows0, sem0, semS0))
            if jj + 1 < G:
                if jj >= 1:
                    pltpu.make_async_copy(
                        nbuf, acc.at[dst_v.at[jj - 1]], nssem).wait()
                pltpu.async_copy(h_hbm.at[src_v.at[jj + 1]], nbuf, ngsem)
            pltpu.make_async_copy(h_hbm.at[src_v.at[jj]], rbuf, gsem).wait()
            pltpu.async_copy(rbuf, acc.at[dst_v.at[jj]], ssem, add=True)
        pltpu.make_async_copy(rows0, acc.at[dst_v.at[G - 2]], semS0).wait()
        pltpu.make_async_copy(rows1, acc.at[dst_v.at[G - 1]], semS1).wait()

    plsc.subcore_barrier()
    pltpu.sync_copy(acc.at[pl.ds(r0, RPS)], agg_hbm.at[cid, pl.ds(r0, RPS)])


def _norms(d0, d1):
    od = d0[0, :, 0:1] + d1[0, :, 0:1]
    idg = d0[0, :, 127:128] + d1[0, :, 127:128]
    return (lax.rsqrt(jnp.maximum(od, 1.0)),
            lax.rsqrt(jnp.maximum(idg, 1.0)))


def _tc_pre_body(x_ref, w_ref, d0_ref, d1_ref, h_ref):
    ns, _ = _norms(d0_ref[...], d1_ref[...])
    h_ref[...] = jnp.dot(x_ref[...], w_ref[...],
                         preferred_element_type=jnp.float32) * ns


def _tc_mid_body(a0_ref, a1_ref, d0_ref, d1_ref, b_ref, w_ref, h_ref):
    ns, nd = _norms(d0_ref[...], d1_ref[...])
    agg = a0_ref[0] + a1_ref[0]
    t = jnp.maximum(agg * nd + b_ref[...], 0.0)
    h_ref[...] = jnp.dot(t * ns, w_ref[...],
                         preferred_element_type=jnp.float32)


def _tc_post_body(a0_ref, a1_ref, d0_ref, d1_ref, b_ref, o_ref):
    _, nd = _norms(d0_ref[...], d1_ref[...])
    o_ref[...] = (a0_ref[0] + a1_ref[0]) * nd + b_ref[...]


def _agg_specs():
    return [pl.BlockSpec((1, R, D), lambda i, c=c: (c, i, 0))
            for c in (0, 1)]


_b_spec = pl.BlockSpec((1, D), lambda i: (0, 0))
_w_spec = pl.BlockSpec((D, D), lambda i: (0, 0))
_row_spec = pl.BlockSpec((R, D), lambda i: (i, 0))


def _tc_pre(x, W1, deg):
    return pl.pallas_call(
        _tc_pre_body,
        grid=(N // R,),
        in_specs=[_row_spec, _w_spec] + _agg_specs(),
        out_specs=_row_spec,
        out_shape=jax.ShapeDtypeStruct((N, D), jnp.float32),
    )(x, W1, deg, deg)


def _tc_mid(agg, deg, b1, W2):
    return pl.pallas_call(
        _tc_mid_body,
        grid=(N // R,),
        in_specs=_agg_specs() + _agg_specs() + [_b_spec, _w_spec],
        out_specs=_row_spec,
        out_shape=jax.ShapeDtypeStruct((N, D), jnp.float32),
    )(agg, agg, deg, deg, b1, W2)


def _tc_post(agg, deg, b2):
    return pl.pallas_call(
        _tc_post_body,
        grid=(N // R,),
        in_specs=_agg_specs() + _agg_specs() + [_b_spec],
        out_specs=_row_spec,
        out_shape=jax.ShapeDtypeStruct((N, D), jnp.float32),
    )(agg, agg, deg, deg, b2)


def kernel(x, edge_index, W1, b1, W2, b2):
    loop_idx = jnp.arange(N, dtype=jnp.int32)
    pad_iota = jnp.arange(PAD, dtype=jnp.int32)
    pad_junk = N + pad_iota % (NP - N)
    pad_src_agg = pad_iota % N
    src4d = jnp.concatenate([edge_index[0], loop_idx, pad_junk]
                            ).reshape(NW, NG, G, C)
    src4 = jnp.concatenate([edge_index[0], loop_idx, pad_src_agg]
                           ).reshape(NW, NG, G, C)
    dst4 = jnp.concatenate([edge_index[1], loop_idx, pad_junk]
                           ).reshape(NW, NG, G, C)
    lane = jnp.arange(D, dtype=jnp.int32)
    vals = jnp.stack([
        jnp.broadcast_to((lane < 64).astype(jnp.float32), (C, D)),
        jnp.broadcast_to((lane >= 64).astype(jnp.float32), (C, D)),
    ])
    z128 = jnp.zeros((NP, D), jnp.float32)

    deg = _sc_degrees(src4d, dst4, vals, z128)
    h1 = _tc_pre(x, W1, deg)
    agg1 = _sc_aggregate(h1, src4, dst4, z128)
    h2 = _tc_mid(agg1, deg, b1.reshape(1, D), W2)
    agg2 = _sc_aggregate(h2, src4, dst4, z128)
    return _tc_post(agg2, deg, b2.reshape(1, D))


# speedup vs baseline: 14.3130x; 1.3026x over previous
import dataclasses
import functools

import jax
import jax.numpy as jnp
from jax import lax
from jax.experimental import pallas as pl
from jax.experimental.pallas import tpu as pltpu
from jax.experimental.pallas import tpu_sc as plsc

N = 10000
NP = 10240
D = 128
NE = 320000
NC, NS = 2, 16
NW = NC * NS
C = 128
EP = NE + N
K = 84
G = 14
NG = K // G
E_PAD = NW * K * C
PAD = E_PAD - EP
RPS = NP // NS
R = 1000

_mesh = plsc.VectorSubcoreMesh(core_axis_name="c", subcore_axis_name="s")


_cp = pltpu.CompilerParams()
if "needs_layout_passes" in pltpu.CompilerParams.__dataclass_fields__:
    _cp = dataclasses.replace(_cp, needs_layout_passes=False)

EXPC = 64


@functools.partial(
    pl.kernel, mesh=_mesh, compiler_params=_cp,
    out_type=jax.ShapeDtypeStruct((NC, NP, D), jnp.float32),
    scratch_types=[
        pltpu.VMEM((G, C), jnp.int32),
        pltpu.VMEM((G, C), jnp.int32),
        pltpu.VMEM((NP,), jnp.float32),
        pltpu.VMEM((NP,), jnp.float32),
        pltpu.VMEM((NS, RPS), jnp.float32),
        pltpu.VMEM((EXPC, D), jnp.float32),
        pltpu.VMEM_SHARED((NS, 2, NP), jnp.float32),
    ],
)
def _sc_degrees(src_hbm, dst_hbm, deg_hbm,
                src_v, dst_v, ho, hi, red, exp, shared):
    cid = lax.axis_index("c")
    sid = lax.axis_index("s")
    wid = cid * NS + sid
    zeros16 = jnp.zeros((16,), jnp.float32)
    ones16 = jnp.ones((16,), jnp.float32)

    @pl.loop(0, NP, step=16)
    def _(i):
        ho[pl.ds(i, 16)] = zeros16
        hi[pl.ds(i, 16)] = zeros16

    @pl.loop(0, NG)
    def _(g):
        pltpu.sync_copy(src_hbm.at[wid, g], src_v)
        pltpu.sync_copy(dst_hbm.at[wid, g], dst_v)
        for jj in range(G):
            for v in range(C // 16):
                plsc.addupdate_scatter(
                    ho, [src_v[jj, pl.ds(v * 16, 16)]], ones16)
                plsc.addupdate_scatter(
                    hi, [dst_v[jj, pl.ds(v * 16, 16)]], ones16)

    pltpu.sync_copy(ho, shared.at[sid, 0])
    pltpu.sync_copy(hi, shared.at[sid, 1])
    plsc.subcore_barrier()
    r0 = sid * RPS
    for h, dest in ((0, ho), (1, hi)):
        for t in range(NS):
            pltpu.sync_copy(shared.at[t, h, pl.ds(r0, RPS)], red.at[t])
        for cc in range(RPS // 16):
            accv = red[0, pl.ds(cc * 16, 16)]
            for t in range(1, NS):
                accv = accv + red[t, pl.ds(cc * 16, 16)]
            dest[pl.ds(cc * 16, 16)] = accv
    iota16 = lax.iota(jnp.int32, 16)
    col0 = jnp.zeros((16,), jnp.int32)
    col127 = jnp.full((16,), 127, jnp.int32)
    for ch in range(RPS // EXPC):
        base = ch * EXPC
        for q in range(EXPC // 16):
            rows16 = iota16 + (q * 16)
            plsc.store_scatter(exp, [rows16, col0],
                               ho[pl.ds(base + q * 16, 16)])
            plsc.store_scatter(exp, [rows16, col127],
                               hi[pl.ds(base + q * 16, 16)])
        pltpu.sync_copy(exp, deg_hbm.at[cid, pl.ds(r0 + base, EXPC)])


@functools.partial(
    pl.kernel, mesh=_mesh,
    out_type=jax.ShapeDtypeStruct((NC, NP, D), jnp.float32),
    scratch_types=[
        pltpu.VMEM((G, C), jnp.int32),
        pltpu.VMEM((G, C), jnp.int32),
        pltpu.VMEM((C, D), jnp.float32),
        pltpu.VMEM((C, D), jnp.float32),
        pltpu.VMEM_SHARED((NP, D), jnp.float32),
        pltpu.SemaphoreType.DMA,
        pltpu.SemaphoreType.DMA,
        pltpu.SemaphoreType.DMA,
        pltpu.SemaphoreType.DMA,
    ],
)
def _sc_aggregate(h_hbm, src_hbm, dst_hbm, z_hbm, agg_hbm,
                  src_v, dst_v, rows0, rows1, acc, sem0, sem1, semS0, semS1):
    cid = lax.axis_index("c")
    sid = lax.axis_index("s")
    wid = cid * NS + sid
    r0 = sid * RPS
    pltpu.sync_copy(z_hbm.at[pl.ds(r0, RPS)], acc.at[pl.ds(r0, RPS)])
    plsc.subcore_barrier()

    @pl.loop(0, NG)
    def _(g):
        pltpu.sync_copy(src_hbm.at[wid, g], src_v)
        pltpu.sync_copy(dst_hbm.at[wid, g], dst_v)
        pltpu.async_copy(h_hbm.at[src_v.at[0]], rows0, sem0)
        for jj in range(G):
            rbuf, gsem, ssem = ((rows0, sem0, semS0) if jj % 2 == 0
                                else (rows1, sem1, semS1))
            nbuf, ngsem, nssem = ((rows1, sem1, semS1) if jj % 2 == 0
                                  else (rows0, sem0, semS0))
            if jj + 1 < G:
                if jj >= 1:
                    pltpu.make_async_copy(
                        nbuf, acc.at[dst_v.at[jj - 1]], nssem).wait()
                pltpu.async_copy(h_hbm.at[src_v.at[jj + 1]], nbuf, ngsem)
            pltpu.make_async_copy(h_hbm.at[src_v.at[jj]], rbuf, gsem).wait()
            pltpu.async_copy(rbuf, acc.at[dst_v.at[jj]], ssem, add=True)
        pltpu.make_async_copy(rows0, acc.at[dst_v.at[G - 2]], semS0).wait()
        pltpu.make_async_copy(rows1, acc.at[dst_v.at[G - 1]], semS1).wait()

    plsc.subcore_barrier()
    pltpu.sync_copy(acc.at[pl.ds(r0, RPS)], agg_hbm.at[cid, pl.ds(r0, RPS)])


def _norms(d0, d1):
    od = d0[0, :, 0:1] + d1[0, :, 0:1]
    idg = d0[0, :, 127:128] + d1[0, :, 127:128]
    return (lax.rsqrt(jnp.maximum(od, 1.0)),
            lax.rsqrt(jnp.maximum(idg, 1.0)))


def _tc_pre_body(x_ref, w_ref, d0_ref, d1_ref, h_ref):
    ns, _ = _norms(d0_ref[...], d1_ref[...])
    h_ref[...] = jnp.dot(x_ref[...], w_ref[...],
                         preferred_element_type=jnp.float32) * ns


def _tc_mid_body(a0_ref, a1_ref, d0_ref, d1_ref, b_ref, w_ref, h_ref):
    ns, nd = _norms(d0_ref[...], d1_ref[...])
    agg = a0_ref[0] + a1_ref[0]
    t = jnp.maximum(agg * nd + b_ref[...], 0.0)
    h_ref[...] = jnp.dot(t * ns, w_ref[...],
                         preferred_element_type=jnp.float32)


def _tc_post_body(a0_ref, a1_ref, d0_ref, d1_ref, b_ref, o_ref):
    _, nd = _norms(d0_ref[...], d1_ref[...])
    o_ref[...] = (a0_ref[0] + a1_ref[0]) * nd + b_ref[...]


def _agg_specs():
    return [pl.BlockSpec((1, R, D), lambda i, c=c: (c, i, 0))
            for c in (0, 1)]


_b_spec = pl.BlockSpec((1, D), lambda i: (0, 0))
_w_spec = pl.BlockSpec((D, D), lambda i: (0, 0))
_row_spec = pl.BlockSpec((R, D), lambda i: (i, 0))


def _tc_pre(x, W1, deg):
    return pl.pallas_call(
        _tc_pre_body,
        grid=(N // R,),
        in_specs=[_row_spec, _w_spec] + _agg_specs(),
        out_specs=_row_spec,
        out_shape=jax.ShapeDtypeStruct((N, D), jnp.float32),
    )(x, W1, deg, deg)


def _tc_mid(agg, deg, b1, W2):
    return pl.pallas_call(
        _tc_mid_body,
        grid=(N // R,),
        in_specs=_agg_specs() + _agg_specs() + [_b_spec, _w_spec],
        out_specs=_row_spec,
        out_shape=jax.ShapeDtypeStruct((N, D), jnp.float32),
    )(agg, agg, deg, deg, b1, W2)


def _tc_post(agg, deg, b2):
    return pl.pallas_call(
        _tc_post_body,
        grid=(N // R,),
        in_specs=_agg_specs() + _agg_specs() + [_b_spec],
        out_specs=_row_spec,
        out_shape=jax.ShapeDtypeStruct((N, D), jnp.float32),
    )(agg, agg, deg, deg, b2)


def kernel(x, edge_index, W1, b1, W2, b2):
    loop_idx = jnp.arange(N, dtype=jnp.int32)
    pad_iota = jnp.arange(PAD, dtype=jnp.int32)
    pad_junk = N + pad_iota % (NP - N)
    pad_src_agg = pad_iota % N
    src4d = jnp.concatenate([edge_index[0], loop_idx, pad_junk]
                            ).reshape(NW, NG, G, C)
    src4 = jnp.concatenate([edge_index[0], loop_idx, pad_src_agg]
                           ).reshape(NW, NG, G, C)
    dst4 = jnp.concatenate([edge_index[1], loop_idx, pad_junk]
                           ).reshape(NW, NG, G, C)
    z128 = jnp.zeros((NP, D), jnp.float32)

    deg = _sc_degrees(src4d, dst4)
    h1 = _tc_pre(x, W1, deg)
    agg1 = _sc_aggregate(h1, src4, dst4, z128)
    h2 = _tc_mid(agg1, deg, b1.reshape(1, D), W2)
    agg2 = _sc_aggregate(h2, src4, dst4, z128)
    return _tc_post(agg2, deg, b2.reshape(1, D))


# speedup vs baseline: 15.2471x; 1.0653x over previous
import dataclasses
import functools

import jax
import jax.numpy as jnp
from jax import lax
from jax.experimental import pallas as pl
from jax.experimental.pallas import tpu as pltpu
from jax.experimental.pallas import tpu_sc as plsc

N = 10000
NP = 10240
D = 128
NE = 320000
NC, NS = 2, 16
NW = NC * NS
C = 128
K = 80
G = 16
NG = K // G
E_PAD = NW * K * C
PAD = E_PAD - NE
RPS = NP // NS
R = 1000

_mesh = plsc.VectorSubcoreMesh(core_axis_name="c", subcore_axis_name="s")


_cp = pltpu.CompilerParams()
if "needs_layout_passes" in pltpu.CompilerParams.__dataclass_fields__:
    _cp = dataclasses.replace(_cp, needs_layout_passes=False)

EXPC = 64


@functools.partial(
    pl.kernel, mesh=_mesh, compiler_params=_cp,
    out_type=jax.ShapeDtypeStruct((NC, NP, D), jnp.float32),
    scratch_types=[
        pltpu.VMEM((G, C), jnp.int32),
        pltpu.VMEM((G, C), jnp.int32),
        pltpu.VMEM((NP,), jnp.float32),
        pltpu.VMEM((NP,), jnp.float32),
        pltpu.VMEM((NS, RPS), jnp.float32),
        pltpu.VMEM((EXPC, D), jnp.float32),
        pltpu.VMEM_SHARED((NS, 2, NP), jnp.float32),
    ],
)
def _sc_degrees(src_hbm, dst_hbm, deg_hbm,
                src_v, dst_v, ho, hi, red, exp, shared):
    cid = lax.axis_index("c")
    sid = lax.axis_index("s")
    wid = cid * NS + sid
    zeros16 = jnp.zeros((16,), jnp.float32)
    ones16 = jnp.ones((16,), jnp.float32)

    @pl.loop(0, NP, step=16)
    def _(i):
        ho[pl.ds(i, 16)] = zeros16
        hi[pl.ds(i, 16)] = zeros16

    @pl.loop(0, NG)
    def _(g):
        pltpu.sync_copy(src_hbm.at[wid, g], src_v)
        pltpu.sync_copy(dst_hbm.at[wid, g], dst_v)
        for jj in range(G):
            for v in range(C // 16):
                plsc.addupdate_scatter(
                    ho, [src_v[jj, pl.ds(v * 16, 16)]], ones16)
                plsc.addupdate_scatter(
                    hi, [dst_v[jj, pl.ds(v * 16, 16)]], ones16)

    pltpu.sync_copy(ho, shared.at[sid, 0])
    pltpu.sync_copy(hi, shared.at[sid, 1])
    plsc.subcore_barrier()
    r0 = sid * RPS
    for h, dest in ((0, ho), (1, hi)):
        for t in range(NS):
            pltpu.sync_copy(shared.at[t, h, pl.ds(r0, RPS)], red.at[t])
        for cc in range(RPS // 16):
            accv = red[0, pl.ds(cc * 16, 16)]
            for t in range(1, NS):
                accv = accv + red[t, pl.ds(cc * 16, 16)]
            dest[pl.ds(cc * 16, 16)] = accv
    iota16 = lax.iota(jnp.int32, 16)
    col0 = jnp.zeros((16,), jnp.int32)
    col127 = jnp.full((16,), 127, jnp.int32)
    for ch in range(RPS // EXPC):
        base = ch * EXPC
        for q in range(EXPC // 16):
            rows16 = iota16 + (q * 16)
            plsc.store_scatter(exp, [rows16, col0],
                               ho[pl.ds(base + q * 16, 16)])
            plsc.store_scatter(exp, [rows16, col127],
                               hi[pl.ds(base + q * 16, 16)])
        pltpu.sync_copy(exp, deg_hbm.at[cid, pl.ds(r0 + base, EXPC)])


@functools.partial(
    pl.kernel, mesh=_mesh,
    out_type=jax.ShapeDtypeStruct((NC, NP, D), jnp.float32),
    scratch_types=[
        pltpu.VMEM((G, C), jnp.int32),
        pltpu.VMEM((G, C), jnp.int32),
        pltpu.VMEM((C, D), jnp.float32),
        pltpu.VMEM((C, D), jnp.float32),
        pltpu.VMEM_SHARED((NP, D), jnp.float32),
        pltpu.SemaphoreType.DMA,
        pltpu.SemaphoreType.DMA,
        pltpu.SemaphoreType.DMA,
        pltpu.SemaphoreType.DMA,
    ],
)
def _sc_aggregate(h_hbm, src_hbm, dst_hbm, z_hbm, agg_hbm,
                  src_v, dst_v, rows0, rows1, acc, sem0, sem1, semS0, semS1):
    cid = lax.axis_index("c")
    sid = lax.axis_index("s")
    wid = cid * NS + sid
    r0 = sid * RPS
    pltpu.sync_copy(z_hbm.at[pl.ds(r0, RPS)], acc.at[pl.ds(r0, RPS)])
    plsc.subcore_barrier()

    @pl.loop(0, NG)
    def _(g):
        pltpu.sync_copy(src_hbm.at[wid, g], src_v)
        pltpu.sync_copy(dst_hbm.at[wid, g], dst_v)
        pltpu.async_copy(h_hbm.at[src_v.at[0]], rows0, sem0)
        for jj in range(G):
            rbuf, gsem, ssem = ((rows0, sem0, semS0) if jj % 2 == 0
                                else (rows1, sem1, semS1))
            nbuf, ngsem, nssem = ((rows1, sem1, semS1) if jj % 2 == 0
                                  else (rows0, sem0, semS0))
            if jj + 1 < G:
                if jj >= 1:
                    pltpu.make_async_copy(
                        nbuf, acc.at[dst_v.at[jj - 1]], nssem).wait()
                pltpu.async_copy(h_hbm.at[src_v.at[jj + 1]], nbuf, ngsem)
            pltpu.make_async_copy(h_hbm.at[src_v.at[jj]], rbuf, gsem).wait()
            pltpu.async_copy(rbuf, acc.at[dst_v.at[jj]], ssem, add=True)
        pltpu.make_async_copy(rows0, acc.at[dst_v.at[G - 2]], semS0).wait()
        pltpu.make_async_copy(rows1, acc.at[dst_v.at[G - 1]], semS1).wait()

    plsc.subcore_barrier()
    pltpu.sync_copy(acc.at[pl.ds(r0, RPS)], agg_hbm.at[cid, pl.ds(r0, RPS)])


def _norms(d0, d1):
    od = d0[0, :, 0:1] + d1[0, :, 0:1] + 1.0
    idg = d0[0, :, 127:128] + d1[0, :, 127:128] + 1.0
    return (lax.rsqrt(jnp.maximum(od, 1.0)),
            lax.rsqrt(jnp.maximum(idg, 1.0)))


def _tc_pre_body(x_ref, w_ref, d0_ref, d1_ref, h_ref):
    ns, _ = _norms(d0_ref[...], d1_ref[...])
    h_ref[...] = jnp.dot(x_ref[...], w_ref[...],
                         preferred_element_type=jnp.float32) * ns


def _tc_mid_body(a0_ref, a1_ref, h1_ref, d0_ref, d1_ref, b_ref, w_ref, h_ref):
    ns, nd = _norms(d0_ref[...], d1_ref[...])
    agg = a0_ref[0] + a1_ref[0] + h1_ref[...]
    t = jnp.maximum(agg * nd + b_ref[...], 0.0)
    h_ref[...] = jnp.dot(t * ns, w_ref[...],
                         preferred_element_type=jnp.float32)


def _tc_post_body(a0_ref, a1_ref, h2_ref, d0_ref, d1_ref, b_ref, o_ref):
    _, nd = _norms(d0_ref[...], d1_ref[...])
    o_ref[...] = (a0_ref[0] + a1_ref[0] + h2_ref[...]) * nd + b_ref[...]


def _agg_specs():
    return [pl.BlockSpec((1, R, D), lambda i, c=c: (c, i, 0))
            for c in (0, 1)]


_b_spec = pl.BlockSpec((1, D), lambda i: (0, 0))
_w_spec = pl.BlockSpec((D, D), lambda i: (0, 0))
_row_spec = pl.BlockSpec((R, D), lambda i: (i, 0))


def _tc_pre(x, W1, deg):
    return pl.pallas_call(
        _tc_pre_body,
        grid=(N // R,),
        in_specs=[_row_spec, _w_spec] + _agg_specs(),
        out_specs=_row_spec,
        out_shape=jax.ShapeDtypeStruct((N, D), jnp.float32),
    )(x, W1, deg, deg)


def _tc_mid(agg, h1, deg, b1, W2):
    return pl.pallas_call(
        _tc_mid_body,
        grid=(N // R,),
        in_specs=(_agg_specs() + [_row_spec] + _agg_specs()
                  + [_b_spec, _w_spec]),
        out_specs=_row_spec,
        out_shape=jax.ShapeDtypeStruct((N, D), jnp.float32),
    )(agg, agg, h1, deg, deg, b1, W2)


def _tc_post(agg, h2, deg, b2):
    return pl.pallas_call(
        _tc_post_body,
        grid=(N // R,),
        in_specs=_agg_specs() + [_row_spec] + _agg_specs() + [_b_spec],
        out_specs=_row_spec,
        out_shape=jax.ShapeDtypeStruct((N, D), jnp.float32),
    )(agg, agg, h2, deg, deg, b2)


def kernel(x, edge_index, W1, b1, W2, b2):
    pad_iota = jnp.arange(PAD, dtype=jnp.int32)
    pad_junk = N + pad_iota % (NP - N)
    pad_src_agg = pad_iota % N
    src4d = jnp.concatenate([edge_index[0], pad_junk]).reshape(NW, NG, G, C)
    src4 = jnp.concatenate([edge_index[0], pad_src_agg]).reshape(NW, NG, G, C)
    dst4 = jnp.concatenate([edge_index[1], pad_junk]).reshape(NW, NG, G, C)
    z128 = jnp.zeros((NP, D), jnp.float32)

    deg = _sc_degrees(src4d, dst4)
    h1 = _tc_pre(x, W1, deg)
    agg1 = _sc_aggregate(h1, src4, dst4, z128)
    h2 = _tc_mid(agg1, h1, deg, b1.reshape(1, D), W2)
    agg2 = _sc_aggregate(h2, src4, dst4, z128)
    return _tc_post(agg2, h2, deg, b2.reshape(1, D))


# speedup vs baseline: 16.0904x; 1.0553x over previous
import dataclasses
import functools

import jax
import jax.numpy as jnp
from jax import lax
from jax.experimental import pallas as pl
from jax.experimental.pallas import tpu as pltpu
from jax.experimental.pallas import tpu_sc as plsc

N = 10000
NP = 10240
D = 128
NE = 320000
NC, NS = 2, 16
NW = NC * NS
C = 128
K = 80
G = 40
NG = K // G
E_PAD = NW * K * C
PAD = E_PAD - NE
RPS = NP // NS
R = 1000

_mesh = plsc.VectorSubcoreMesh(core_axis_name="c", subcore_axis_name="s")


_cp = pltpu.CompilerParams()
if "needs_layout_passes" in pltpu.CompilerParams.__dataclass_fields__:
    _cp = dataclasses.replace(_cp, needs_layout_passes=False)

EXPC = 64


@functools.partial(
    pl.kernel, mesh=_mesh, compiler_params=_cp,
    out_type=jax.ShapeDtypeStruct((NC, NP, D), jnp.float32),
    scratch_types=[
        pltpu.VMEM((G, C), jnp.int32),
        pltpu.VMEM((G, C), jnp.int32),
        pltpu.VMEM((NP,), jnp.float32),
        pltpu.VMEM((NP,), jnp.float32),
        pltpu.VMEM((NS, RPS), jnp.float32),
        pltpu.VMEM((EXPC, D), jnp.float32),
        pltpu.VMEM_SHARED((NS, 2, NP), jnp.float32),
    ],
)
def _sc_degrees(src_hbm, dst_hbm, deg_hbm,
                src_v, dst_v, ho, hi, red, exp, shared):
    cid = lax.axis_index("c")
    sid = lax.axis_index("s")
    wid = cid * NS + sid
    zeros16 = jnp.zeros((16,), jnp.float32)
    ones16 = jnp.ones((16,), jnp.float32)

    @pl.loop(0, NP, step=16)
    def _(i):
        ho[pl.ds(i, 16)] = zeros16
        hi[pl.ds(i, 16)] = zeros16

    @pl.loop(0, NG)
    def _(g):
        pltpu.sync_copy(src_hbm.at[wid, g], src_v)
        pltpu.sync_copy(dst_hbm.at[wid, g], dst_v)
        for jj in range(G):
            for v in range(C // 16):
                plsc.addupdate_scatter(
                    ho, [src_v[jj, pl.ds(v * 16, 16)]], ones16)
                plsc.addupdate_scatter(
                    hi, [dst_v[jj, pl.ds(v * 16, 16)]], ones16)

    pltpu.sync_copy(ho, shared.at[sid, 0])
    pltpu.sync_copy(hi, shared.at[sid, 1])
    plsc.subcore_barrier()
    r0 = sid * RPS
    for h, dest in ((0, ho), (1, hi)):
        for t in range(NS):
            pltpu.sync_copy(shared.at[t, h, pl.ds(r0, RPS)], red.at[t])
        for cc in range(RPS // 16):
            accv = red[0, pl.ds(cc * 16, 16)]
            for t in range(1, NS):
                accv = accv + red[t, pl.ds(cc * 16, 16)]
            dest[pl.ds(cc * 16, 16)] = accv
    iota16 = lax.iota(jnp.int32, 16)
    col0 = jnp.zeros((16,), jnp.int32)
    col127 = jnp.full((16,), 127, jnp.int32)
    for ch in range(RPS // EXPC):
        base = ch * EXPC
        for q in range(EXPC // 16):
            rows16 = iota16 + (q * 16)
            plsc.store_scatter(exp, [rows16, col0],
                               ho[pl.ds(base + q * 16, 16)])
            plsc.store_scatter(exp, [rows16, col127],
                               hi[pl.ds(base + q * 16, 16)])
        pltpu.sync_copy(exp, deg_hbm.at[cid, pl.ds(r0 + base, EXPC)])


@functools.partial(
    pl.kernel, mesh=_mesh,
    out_type=jax.ShapeDtypeStruct((NC, NP, D), jnp.float32),
    scratch_types=[
        pltpu.VMEM((G, C), jnp.int32),
        pltpu.VMEM((G, C), jnp.int32),
        pltpu.VMEM((C, D), jnp.float32),
        pltpu.VMEM((C, D), jnp.float32),
        pltpu.VMEM_SHARED((NP, D), jnp.float32),
        pltpu.SemaphoreType.DMA,
        pltpu.SemaphoreType.DMA,
        pltpu.SemaphoreType.DMA,
        pltpu.SemaphoreType.DMA,
    ],
)
def _sc_aggregate(h_hbm, src_hbm, dst_hbm, z_hbm, agg_hbm,
                  src_v, dst_v, rows0, rows1, acc, sem0, sem1, semS0, semS1):
    cid = lax.axis_index("c")
    sid = lax.axis_index("s")
    wid = cid * NS + sid
    r0 = sid * RPS
    pltpu.sync_copy(z_hbm.at[pl.ds(r0, RPS)], acc.at[pl.ds(r0, RPS)])
    plsc.subcore_barrier()

    @pl.loop(0, NG)
    def _(g):
        pltpu.sync_copy(src_hbm.at[wid, g], src_v)
        pltpu.sync_copy(dst_hbm.at[wid, g], dst_v)
        pltpu.async_copy(h_hbm.at[src_v.at[0]], rows0, sem0)
        for jj in range(G):
            rbuf, gsem, ssem = ((rows0, sem0, semS0) if jj % 2 == 0
                                else (rows1, sem1, semS1))
            nbuf, ngsem, nssem = ((rows1, sem1, semS1) if jj % 2 == 0
                                  else (rows0, sem0, semS0))
            if jj + 1 < G:
                if jj >= 1:
                    pltpu.make_async_copy(
                        nbuf, acc.at[dst_v.at[jj - 1]], nssem).wait()
                pltpu.async_copy(h_hbm.at[src_v.at[jj + 1]], nbuf, ngsem)
            pltpu.make_async_copy(h_hbm.at[src_v.at[jj]], rbuf, gsem).wait()
            pltpu.async_copy(rbuf, acc.at[dst_v.at[jj]], ssem, add=True)
        pltpu.make_async_copy(rows0, acc.at[dst_v.at[G - 2]], semS0).wait()
        pltpu.make_async_copy(rows1, acc.at[dst_v.at[G - 1]], semS1).wait()

    plsc.subcore_barrier()
    pltpu.sync_copy(acc.at[pl.ds(r0, RPS)], agg_hbm.at[cid, pl.ds(r0, RPS)])


def _norms(d0, d1):
    od = d0[0, :, 0:1] + d1[0, :, 0:1] + 1.0
    idg = d0[0, :, 127:128] + d1[0, :, 127:128] + 1.0
    return (lax.rsqrt(jnp.maximum(od, 1.0)),
            lax.rsqrt(jnp.maximum(idg, 1.0)))


def _tc_pre_body(x_ref, w_ref, d0_ref, d1_ref, h_ref):
    ns, _ = _norms(d0_ref[...], d1_ref[...])
    h_ref[...] = jnp.dot(x_ref[...], w_ref[...],
                         preferred_element_type=jnp.float32) * ns


def _tc_mid_body(a0_ref, a1_ref, h1_ref, d0_ref, d1_ref, b_ref, w_ref, h_ref):
    ns, nd = _norms(d0_ref[...], d1_ref[...])
    agg = a0_ref[0] + a1_ref[0] + h1_ref[...]
    t = jnp.maximum(agg * nd + b_ref[...], 0.0)
    h_ref[...] = jnp.dot(t * ns, w_ref[...],
                         preferred_element_type=jnp.float32)


def _tc_post_body(a0_ref, a1_ref, h2_ref, d0_ref, d1_ref, b_ref, o_ref):
    _, nd = _norms(d0_ref[...], d1_ref[...])
    o_ref[...] = (a0_ref[0] + a1_ref[0] + h2_ref[...]) * nd + b_ref[...]


def _agg_specs():
    return [pl.BlockSpec((1, R, D), lambda i, c=c: (c, i, 0))
            for c in (0, 1)]


_b_spec = pl.BlockSpec((1, D), lambda i: (0, 0))
_w_spec = pl.BlockSpec((D, D), lambda i: (0, 0))
_row_spec = pl.BlockSpec((R, D), lambda i: (i, 0))


def _tc_pre(x, W1, deg):
    return pl.pallas_call(
        _tc_pre_body,
        grid=(N // R,),
        in_specs=[_row_spec, _w_spec] + _agg_specs(),
        out_specs=_row_spec,
        out_shape=jax.ShapeDtypeStruct((N, D), jnp.float32),
    )(x, W1, deg, deg)


def _tc_mid(agg, h1, deg, b1, W2):
    return pl.pallas_call(
        _tc_mid_body,
        grid=(N // R,),
        in_specs=(_agg_specs() + [_row_spec] + _agg_specs()
                  + [_b_spec, _w_spec]),
        out_specs=_row_spec,
        out_shape=jax.ShapeDtypeStruct((N, D), jnp.float32),
    )(agg, agg, h1, deg, deg, b1, W2)


def _tc_post(agg, h2, deg, b2):
    return pl.pallas_call(
        _tc_post_body,
        grid=(N // R,),
        in_specs=_agg_specs() + [_row_spec] + _agg_specs() + [_b_spec],
        out_specs=_row_spec,
        out_shape=jax.ShapeDtypeStruct((N, D), jnp.float32),
    )(agg, agg, h2, deg, deg, b2)


def kernel(x, edge_index, W1, b1, W2, b2):
    pad_iota = jnp.arange(PAD, dtype=jnp.int32)
    pad_junk = N + pad_iota % (NP - N)
    pad_src_agg = pad_iota % N
    src4d = jnp.concatenate([edge_index[0], pad_junk]).reshape(NW, NG, G, C)
    src4 = jnp.concatenate([edge_index[0], pad_src_agg]).reshape(NW, NG, G, C)
    dst4 = jnp.concatenate([edge_index[1], pad_junk]).reshape(NW, NG, G, C)
    z128 = jnp.zeros((NP, D), jnp.float32)

    deg = _sc_degrees(src4d, dst4)
    h1 = _tc_pre(x, W1, deg)
    agg1 = _sc_aggregate(h1, src4, dst4, z128)
    h2 = _tc_mid(agg1, h1, deg, b1.reshape(1, D), W2)
    agg2 = _sc_aggregate(h2, src4, dst4, z128)
    return _tc_post(agg2, h2, deg, b2.reshape(1, D))
